# baseline (speedup 1.0000x reference)
import jax
import jax.numpy as jnp
from jax.experimental import pallas as pl

FANOUT = 32
D = 128
BLK = 400


def _body(src_ref, dst_ref, w1_ref, w2_ref, b_ref, out_ref):
    agg = dst_ref[...].sum(axis=1)
    out_ref[...] = (
        jnp.dot(src_ref[...], w1_ref[...], preferred_element_type=jnp.float32)
        + jnp.dot(agg, w2_ref[...], preferred_element_type=jnp.float32)
        + b_ref[...]
    )


def kernel(src_feature, dst_feature, W, b):
    n = src_feature.shape[0]
    w1 = W[:D]
    w2 = W[D:] * (1.0 / FANOUT)
    b2 = b.reshape(1, D)
    return pl.pallas_call(
        _body,
        grid=(pl.cdiv(n, BLK),),
        in_specs=[
            pl.BlockSpec((BLK, D), lambda i: (i, 0)),
            pl.BlockSpec((BLK, FANOUT, D), lambda i: (i, 0, 0)),
            pl.BlockSpec((D, D), lambda i: (0, 0)),
            pl.BlockSpec((D, D), lambda i: (0, 0)),
            pl.BlockSpec((1, D), lambda i: (0, 0)),
        ],
        out_specs=pl.BlockSpec((BLK, D), lambda i: (i, 0)),
        out_shape=jax.ShapeDtypeStruct((n, D), jnp.float32),
    )(src_feature, dst_feature, w1, w2, b2)
